# initial kernel scaffold (unmeasured)
import jax
import jax.numpy as jnp
from jax import lax
from jax.experimental import pallas as pl
from jax.experimental.pallas import tpu as pltpu


def kernel(
    x,
):
    def body(*refs):
        pass

    out_shape = jax.ShapeDtypeStruct(..., jnp.float32)
    return pl.pallas_call(body, out_shape=out_shape)(...)


# baseline (device time: 536167 ns/iter reference)
import jax
import jax.numpy as jnp
from jax import lax
from jax.experimental import pallas as pl
from jax.experimental.pallas import tpu as pltpu


def kernel(x):
    M, N = x.shape
    NCOL = N // 2
    H = M // 2
    M_OUT = 2 * M

    def body(x_ref, out_ref, send1, recv1, send2, recv2, copy_sem):
        mx = lax.axis_index("x")
        my = lax.axis_index("y")

        barrier = pltpu.get_barrier_semaphore()
        pl.semaphore_signal(
            barrier, inc=1,
            device_id=(1 - mx, my), device_id_type=pl.DeviceIdType.MESH,
        )
        pl.semaphore_signal(
            barrier, inc=1,
            device_id=(mx, 1 - my), device_id_type=pl.DeviceIdType.MESH,
        )
        pl.semaphore_wait(barrier, 2)

        local = pltpu.make_async_copy(
            x_ref.at[:, pl.ds(mx * NCOL, NCOL)],
            out_ref.at[pl.ds(mx * M, M), :],
            copy_sem,
        )
        local.start()

        p1 = pltpu.make_async_remote_copy(
            src_ref=x_ref.at[pl.ds(my * H, H), pl.ds((1 - mx) * NCOL, NCOL)],
            dst_ref=out_ref.at[pl.ds(mx * M + my * H, H), :],
            send_sem=send1,
            recv_sem=recv1,
            device_id=(1 - mx, my),
            device_id_type=pl.DeviceIdType.MESH,
        )
        p1.start()
        p1.wait()

        recv_row = (1 - mx) * M + my * H
        p2 = pltpu.make_async_remote_copy(
            src_ref=out_ref.at[pl.ds(recv_row, H), :],
            dst_ref=out_ref.at[pl.ds(recv_row, H), :],
            send_sem=send2,
            recv_sem=recv2,
            device_id=(mx, 1 - my),
            device_id_type=pl.DeviceIdType.MESH,
        )
        p2.start()
        p2.wait()
        local.wait()

    return pl.pallas_call(
        body,
        out_shape=jax.ShapeDtypeStruct((M_OUT, NCOL), jnp.float32),
        in_specs=[pl.BlockSpec(memory_space=pl.ANY)],
        out_specs=pl.BlockSpec(memory_space=pl.ANY),
        scratch_shapes=[
            pltpu.SemaphoreType.DMA,
            pltpu.SemaphoreType.DMA,
            pltpu.SemaphoreType.DMA,
            pltpu.SemaphoreType.DMA,
            pltpu.SemaphoreType.DMA,
        ],
        compiler_params=pltpu.CompilerParams(collective_id=0),
    )(x)


# device time: 138257 ns/iter; 3.8780x vs baseline; 3.8780x over previous
import jax
import jax.numpy as jnp
from jax import lax
from jax.experimental import pallas as pl
from jax.experimental.pallas import tpu as pltpu

K = 16
LC = 8


def kernel(x):
    M, N = x.shape
    NCOL = N // 2
    H = M // 2
    M_OUT = 2 * M
    CH = H // K
    LCH = M // LC

    def body(x_ref, out_ref, vrecv1, vrecv2, vstage,
             p1_send, p1_recv, p2_send, p2_recv,
             st1_sems, st2_sems, lc_in_sems, lc_out_sems):
        mx = lax.axis_index("x")
        my = lax.axis_index("y")

        barrier = pltpu.get_barrier_semaphore()
        pl.semaphore_signal(barrier, inc=1, device_id=(1 - mx, my),
                            device_id_type=pl.DeviceIdType.MESH)
        pl.semaphore_signal(barrier, inc=1, device_id=(mx, 1 - my),
                            device_id_type=pl.DeviceIdType.MESH)
        pl.semaphore_wait(barrier, 2)

        p1 = []
        for k in range(K):
            r = pltpu.make_async_remote_copy(
                src_ref=x_ref.at[pl.ds(my * H + k * CH, CH),
                                 pl.ds((1 - mx) * NCOL, NCOL)],
                dst_ref=vrecv1.at[pl.ds(k * CH, CH), :],
                send_sem=p1_send.at[k],
                recv_sem=p1_recv.at[k],
                device_id=(1 - mx, my),
                device_id_type=pl.DeviceIdType.MESH,
            )
            r.start()
            p1.append(r)

        lc_in = [None] * LC
        lc_out = [None] * LC
        for k in range(LC):
            if k >= 2:
                lc_out[k - 2].wait()
            lc_in[k] = pltpu.make_async_copy(
                x_ref.at[pl.ds(k * LCH, LCH), pl.ds(mx * NCOL, NCOL)],
                vstage.at[k % 2],
                lc_in_sems.at[k % 2],
            )
            lc_in[k].start()
            lc_in[k].wait()
            lc_out[k] = pltpu.make_async_copy(
                vstage.at[k % 2],
                out_ref.at[pl.ds(mx * M + k * LCH, LCH), :],
                lc_out_sems.at[k % 2],
            )
            lc_out[k].start()

        p2 = []
        st1 = []
        for k in range(K):
            p1[k].wait_recv()
            r = pltpu.make_async_remote_copy(
                src_ref=vrecv1.at[pl.ds(k * CH, CH), :],
                dst_ref=vrecv2.at[pl.ds(k * CH, CH), :],
                send_sem=p2_send.at[k],
                recv_sem=p2_recv.at[k],
                device_id=(mx, 1 - my),
                device_id_type=pl.DeviceIdType.MESH,
            )
            r.start()
            p2.append(r)
            s = pltpu.make_async_copy(
                vrecv1.at[pl.ds(k * CH, CH), :],
                out_ref.at[pl.ds((1 - mx) * M + my * H + k * CH, CH), :],
                st1_sems.at[k],
            )
            s.start()
            st1.append(s)

        st2 = []
        for k in range(K):
            p2[k].wait_recv()
            s = pltpu.make_async_copy(
                vrecv2.at[pl.ds(k * CH, CH), :],
                out_ref.at[pl.ds((1 - mx) * M + (1 - my) * H + k * CH, CH), :],
                st2_sems.at[k],
            )
            s.start()
            st2.append(s)

        for k in range(K):
            p1[k].wait_send()
            p2[k].wait_send()
            st1[k].wait()
            st2[k].wait()
        lc_out[LC - 2].wait()
        lc_out[LC - 1].wait()

    return pl.pallas_call(
        body,
        out_shape=jax.ShapeDtypeStruct((M_OUT, NCOL), jnp.float32),
        in_specs=[pl.BlockSpec(memory_space=pl.ANY)],
        out_specs=pl.BlockSpec(memory_space=pl.ANY),
        scratch_shapes=[
            pltpu.VMEM((H, NCOL), jnp.float32),
            pltpu.VMEM((H, NCOL), jnp.float32),
            pltpu.VMEM((2, LCH, NCOL), jnp.float32),
            pltpu.SemaphoreType.DMA((K,)),
            pltpu.SemaphoreType.DMA((K,)),
            pltpu.SemaphoreType.DMA((K,)),
            pltpu.SemaphoreType.DMA((K,)),
            pltpu.SemaphoreType.DMA((K,)),
            pltpu.SemaphoreType.DMA((K,)),
            pltpu.SemaphoreType.DMA((2,)),
            pltpu.SemaphoreType.DMA((2,)),
        ],
        compiler_params=pltpu.CompilerParams(collective_id=0),
    )(x)


# device time: 128140 ns/iter; 4.1842x vs baseline; 1.0790x over previous
import jax
import jax.numpy as jnp
from jax import lax
from jax.experimental import pallas as pl
from jax.experimental.pallas import tpu as pltpu

K = 16
LC = 8


def kernel(x):
    M, N = x.shape
    NCOL = N // 2
    H = M // 2
    M_OUT = 2 * M
    CH = H // K
    LCH = M // LC

    def body(x_ref, out_ref, vrecv1, vrecv2, vstage,
             p1_send, p1_recv, p2_send, p2_recv,
             st1_sems, st2_sems, lc_in_sems, lc_out_sems):
        mx = lax.axis_index("x")
        my = lax.axis_index("y")

        barrier = pltpu.get_barrier_semaphore()
        pl.semaphore_signal(barrier, inc=1, device_id=(1 - mx, my),
                            device_id_type=pl.DeviceIdType.MESH)
        pl.semaphore_signal(barrier, inc=1, device_id=(mx, 1 - my),
                            device_id_type=pl.DeviceIdType.MESH)
        pl.semaphore_wait(barrier, 2)

        p1 = []
        for k in range(K):
            r = pltpu.make_async_remote_copy(
                src_ref=x_ref.at[pl.ds(my * H + k * CH, CH),
                                 pl.ds((1 - mx) * NCOL, NCOL)],
                dst_ref=vrecv1.at[pl.ds(k * CH, CH), :],
                send_sem=p1_send.at[k],
                recv_sem=p1_recv.at[k],
                device_id=(1 - mx, my),
                device_id_type=pl.DeviceIdType.MESH,
            )
            r.start()
            p1.append(r)

        lc_in = []
        for k in range(LC):
            c = pltpu.make_async_copy(
                x_ref.at[pl.ds(k * LCH, LCH), pl.ds(mx * NCOL, NCOL)],
                vstage.at[k],
                lc_in_sems.at[k],
            )
            c.start()
            lc_in.append(c)

        p2 = []
        st1 = []
        for k in range(K):
            p1[k].wait_recv()
            r = pltpu.make_async_remote_copy(
                src_ref=vrecv1.at[pl.ds(k * CH, CH), :],
                dst_ref=vrecv2.at[pl.ds(k * CH, CH), :],
                send_sem=p2_send.at[k],
                recv_sem=p2_recv.at[k],
                device_id=(mx, 1 - my),
                device_id_type=pl.DeviceIdType.MESH,
            )
            r.start()
            p2.append(r)
            s = pltpu.make_async_copy(
                vrecv1.at[pl.ds(k * CH, CH), :],
                out_ref.at[pl.ds((1 - mx) * M + my * H + k * CH, CH), :],
                st1_sems.at[k],
            )
            s.start()
            st1.append(s)

        lc_out = []
        for k in range(LC):
            lc_in[k].wait()
            c = pltpu.make_async_copy(
                vstage.at[k],
                out_ref.at[pl.ds(mx * M + k * LCH, LCH), :],
                lc_out_sems.at[k],
            )
            c.start()
            lc_out.append(c)

        st2 = []
        for k in range(K):
            p2[k].wait_recv()
            s = pltpu.make_async_copy(
                vrecv2.at[pl.ds(k * CH, CH), :],
                out_ref.at[pl.ds((1 - mx) * M + (1 - my) * H + k * CH, CH), :],
                st2_sems.at[k],
            )
            s.start()
            st2.append(s)

        for k in range(K):
            p1[k].wait_send()
            p2[k].wait_send()
            st1[k].wait()
            st2[k].wait()
        for k in range(LC):
            lc_out[k].wait()

    return pl.pallas_call(
        body,
        out_shape=jax.ShapeDtypeStruct((M_OUT, NCOL), jnp.float32),
        in_specs=[pl.BlockSpec(memory_space=pl.ANY)],
        out_specs=pl.BlockSpec(memory_space=pl.ANY),
        scratch_shapes=[
            pltpu.VMEM((H, NCOL), jnp.float32),
            pltpu.VMEM((H, NCOL), jnp.float32),
            pltpu.VMEM((LC, LCH, NCOL), jnp.float32),
            pltpu.SemaphoreType.DMA((K,)),
            pltpu.SemaphoreType.DMA((K,)),
            pltpu.SemaphoreType.DMA((K,)),
            pltpu.SemaphoreType.DMA((K,)),
            pltpu.SemaphoreType.DMA((K,)),
            pltpu.SemaphoreType.DMA((K,)),
            pltpu.SemaphoreType.DMA((LC,)),
            pltpu.SemaphoreType.DMA((LC,)),
        ],
        compiler_params=pltpu.CompilerParams(collective_id=0),
    )(x)
